# baseline (speedup 1.0000x reference)
import functools

import jax
import jax.numpy as jnp
from jax import lax
from jax.experimental import pallas as pl
from jax.experimental.pallas import tpu as pltpu
from jax.experimental.pallas import tpu_sc as plsc

NUM_ACTIONS = 100000
EMBED_DIM = 64
HIDDEN_DIM = 256
BATCH = 16384
SEQ = 50

ROW_BLOCK = 2000

NC = 2
NS = 16
NW = NC * NS
B_PER_W = BATCH // NW
CHUNK = 128
CHUNKS_PER_S = B_PER_W // CHUNK
N_CHUNKS = SEQ * CHUNKS_PER_S
PACK_B = BATCH // 2


def _mlp_block(table_ref, w1_ref, b1_ref, w2_ref, b2_ref, out_ref):
    t = table_ref[...].astype(jnp.bfloat16)
    h = (
        jnp.dot(t, w1_ref[...].astype(jnp.bfloat16), preferred_element_type=jnp.float32)
        + b1_ref[...]
    )
    h = h * jax.nn.sigmoid(h)
    out_ref[...] = (
        jnp.dot(
            h.astype(jnp.bfloat16),
            w2_ref[...].astype(jnp.bfloat16),
            preferred_element_type=jnp.float32,
        )
        + b2_ref[...]
    )


def _fuse_table(table, W1, b1, W2, b2):
    grid = (NUM_ACTIONS // ROW_BLOCK,)
    return pl.pallas_call(
        _mlp_block,
        grid=grid,
        in_specs=[
            pl.BlockSpec((ROW_BLOCK, HIDDEN_DIM), lambda i: (i, 0)),
            pl.BlockSpec((HIDDEN_DIM, HIDDEN_DIM), lambda i: (0, 0)),
            pl.BlockSpec((1, HIDDEN_DIM), lambda i: (0, 0)),
            pl.BlockSpec((HIDDEN_DIM, EMBED_DIM), lambda i: (0, 0)),
            pl.BlockSpec((1, EMBED_DIM), lambda i: (0, 0)),
        ],
        out_specs=pl.BlockSpec((ROW_BLOCK, EMBED_DIM), lambda i: (i, 0)),
        out_shape=jax.ShapeDtypeStruct((NUM_ACTIONS, EMBED_DIM), jnp.float32),
    )(table, W1, b1.reshape(1, HIDDEN_DIM), W2, b2.reshape(1, EMBED_DIM))


def _gather_body(fused_hbm, idxt_hbm, out_hbm, idx_v, rows_v, sem_a, sem_b):
    wid = lax.axis_index("s") * NC + lax.axis_index("c")
    b0 = wid * B_PER_W
    pltpu.sync_copy(idxt_hbm.at[:, pl.ds(b0, B_PER_W)], idx_v)

    def fire(c, slot, sem):
        s = c // CHUNKS_PER_S
        j = lax.rem(c, CHUNKS_PER_S)
        pltpu.async_copy(
            fused_hbm.at[idx_v.at[s, pl.ds(j * CHUNK, CHUNK)]],
            rows_v.at[slot],
            sem,
        )

    def drain_and_write(c, slot, sem):
        pltpu.make_async_copy(
            fused_hbm.at[idx_v.at[0, pl.ds(0, CHUNK)]], rows_v.at[slot], sem
        ).wait()
        s = c // CHUNKS_PER_S
        j = lax.rem(c, CHUNKS_PER_S)
        p0 = wid * (B_PER_W // 2) + lax.rem(j, 2) * CHUNK
        col = (j // 2) * EMBED_DIM
        pltpu.sync_copy(
            rows_v.at[slot],
            out_hbm.at[s, pl.ds(p0, CHUNK), pl.ds(col, EMBED_DIM)],
        )

    fire(0, 0, sem_a)

    def step(c, _):
        even = lax.rem(c, 2) == 0
        more = c + 1 < N_CHUNKS

        @pl.when(jnp.logical_and(even, more))
        def _():
            fire(c + 1, 1, sem_b)

        @pl.when(jnp.logical_and(jnp.logical_not(even), more))
        def _():
            fire(c + 1, 0, sem_a)

        @pl.when(even)
        def _():
            drain_and_write(c, 0, sem_a)

        @pl.when(jnp.logical_not(even))
        def _():
            drain_and_write(c, 1, sem_b)

        return 0

    lax.fori_loop(0, N_CHUNKS, step, 0)


@jax.jit
def _sc_gather(fused, idxt):
    mesh = plsc.VectorSubcoreMesh(core_axis_name="c", subcore_axis_name="s")
    return pl.kernel(
        _gather_body,
        out_type=jax.ShapeDtypeStruct((SEQ, PACK_B, 2 * EMBED_DIM), jnp.float32),
        mesh=mesh,
        compiler_params=pltpu.CompilerParams(use_tc_tiling_on_sc=False),
        scratch_types=[
            pltpu.VMEM((SEQ, B_PER_W), jnp.int32),
            pltpu.VMEM((2, CHUNK, EMBED_DIM), jnp.float32),
            pltpu.SemaphoreType.DMA,
            pltpu.SemaphoreType.DMA,
        ],
    )(fused, idxt)


UNPACK_P = 256


def _unpack_block(packed_ref, out_ref):
    x = packed_ref[...]
    lo = x[:, :, :EMBED_DIM].transpose(0, 2, 1)
    hi = x[:, :, EMBED_DIM:].transpose(0, 2, 1)
    out_ref[...] = jnp.concatenate([lo, hi], axis=2)


def _unpack(packed):
    return pl.pallas_call(
        _unpack_block,
        grid=(PACK_B // UNPACK_P,),
        in_specs=[pl.BlockSpec((SEQ, UNPACK_P, 128), lambda i: (0, i, 0))],
        out_specs=pl.BlockSpec(
            (SEQ, EMBED_DIM, 2 * UNPACK_P), lambda i: (0, 0, i)
        ),
        out_shape=jax.ShapeDtypeStruct((SEQ, EMBED_DIM, BATCH), jnp.float32),
    )(packed)


def kernel(action_indices, table, W1, b1, W2, b2):
    idxt = action_indices.astype(jnp.int32).T
    fused = _fuse_table(table, W1, b1, W2, b2)
    return jnp.transpose(_unpack(_sc_gather(fused, idxt)), (2, 0, 1))


# speedup vs baseline: 1.0494x; 1.0494x over previous
import functools

import jax
import jax.numpy as jnp
from jax import lax
from jax.experimental import pallas as pl
from jax.experimental.pallas import tpu as pltpu
from jax.experimental.pallas import tpu_sc as plsc

NUM_ACTIONS = 100000
EMBED_DIM = 64
HIDDEN_DIM = 256
BATCH = 16384
SEQ = 50

ROW_BLOCK = 2000

NC = 2
NS = 16
NW = NC * NS
B_PER_W = BATCH // NW
CHUNK = 128
CHUNKS_PER_S = B_PER_W // CHUNK
N_CHUNKS = SEQ * CHUNKS_PER_S
PACK_B = BATCH // 2


def _mlp_block(table_ref, w1_ref, b1_ref, w2_ref, b2_ref, out_ref):
    t = table_ref[...].astype(jnp.bfloat16)
    h = (
        jnp.dot(t, w1_ref[...].astype(jnp.bfloat16), preferred_element_type=jnp.float32)
        + b1_ref[...]
    )
    h = h * jax.nn.sigmoid(h)
    out_ref[...] = (
        jnp.dot(
            h.astype(jnp.bfloat16),
            w2_ref[...].astype(jnp.bfloat16),
            preferred_element_type=jnp.float32,
        )
        + b2_ref[...]
    )


def _fuse_table(table, W1, b1, W2, b2):
    grid = (NUM_ACTIONS // ROW_BLOCK,)
    return pl.pallas_call(
        _mlp_block,
        grid=grid,
        in_specs=[
            pl.BlockSpec((ROW_BLOCK, HIDDEN_DIM), lambda i: (i, 0)),
            pl.BlockSpec((HIDDEN_DIM, HIDDEN_DIM), lambda i: (0, 0)),
            pl.BlockSpec((1, HIDDEN_DIM), lambda i: (0, 0)),
            pl.BlockSpec((HIDDEN_DIM, EMBED_DIM), lambda i: (0, 0)),
            pl.BlockSpec((1, EMBED_DIM), lambda i: (0, 0)),
        ],
        out_specs=pl.BlockSpec((ROW_BLOCK, EMBED_DIM), lambda i: (i, 0)),
        out_shape=jax.ShapeDtypeStruct((NUM_ACTIONS, EMBED_DIM), jnp.float32),
    )(table, W1, b1.reshape(1, HIDDEN_DIM), W2, b2.reshape(1, EMBED_DIM))


def _gather_body(
    fused_hbm, idxt_hbm, out_hbm, idx_v, rows_v, sem_a, sem_b, sem_c, sem_d
):
    wid = lax.axis_index("s") * NC + lax.axis_index("c")
    b0 = wid * B_PER_W
    pltpu.sync_copy(idxt_hbm.at[:, pl.ds(b0, B_PER_W)], idx_v)

    def fire(c, slot, sem):
        s = c // CHUNKS_PER_S
        j = lax.rem(c, CHUNKS_PER_S)
        pltpu.async_copy(
            fused_hbm.at[idx_v.at[s, pl.ds(j * CHUNK, CHUNK)]],
            rows_v.at[slot],
            sem,
        )

    def drain_and_write(c, slot, sem):
        pltpu.make_async_copy(
            fused_hbm.at[idx_v.at[0, pl.ds(0, CHUNK)]], rows_v.at[slot], sem
        ).wait()
        s = c // CHUNKS_PER_S
        j = lax.rem(c, CHUNKS_PER_S)
        p0 = wid * (B_PER_W // 2) + lax.rem(j, 2) * CHUNK
        col = (j // 2) * EMBED_DIM
        pltpu.sync_copy(
            rows_v.at[slot],
            out_hbm.at[s, pl.ds(p0, CHUNK), pl.ds(col, EMBED_DIM)],
        )

    sems = (sem_a, sem_b, sem_c, sem_d)
    fire(0, 0, sems[0])
    fire(1, 1, sems[1])

    def step(c, _):
        phase = lax.rem(c, 4)

        for k in range(4):

            @pl.when(jnp.logical_and(phase == k, c + 2 < N_CHUNKS))
            def _(k=k):
                fire(c + 2, (k + 2) % 4, sems[(k + 2) % 4])

        for k in range(4):

            @pl.when(phase == k)
            def _(k=k):
                drain_and_write(c, k, sems[k])

        return 0

    lax.fori_loop(0, N_CHUNKS, step, 0)


@jax.jit
def _sc_gather(fused, idxt):
    mesh = plsc.VectorSubcoreMesh(core_axis_name="c", subcore_axis_name="s")
    return pl.kernel(
        _gather_body,
        out_type=jax.ShapeDtypeStruct((SEQ, PACK_B, 2 * EMBED_DIM), jnp.float32),
        mesh=mesh,
        compiler_params=pltpu.CompilerParams(use_tc_tiling_on_sc=False),
        scratch_types=[
            pltpu.VMEM((SEQ, B_PER_W), jnp.int32),
            pltpu.VMEM((4, CHUNK, EMBED_DIM), jnp.float32),
            pltpu.SemaphoreType.DMA,
            pltpu.SemaphoreType.DMA,
            pltpu.SemaphoreType.DMA,
            pltpu.SemaphoreType.DMA,
        ],
    )(fused, idxt)


UNPACK_P = 256


def _unpack_block(packed_ref, out_ref):
    x = packed_ref[...]
    lo = x[:, :, :EMBED_DIM].transpose(0, 2, 1)
    hi = x[:, :, EMBED_DIM:].transpose(0, 2, 1)
    out_ref[...] = jnp.concatenate([lo, hi], axis=2)


def _unpack(packed):
    return pl.pallas_call(
        _unpack_block,
        grid=(PACK_B // UNPACK_P,),
        in_specs=[pl.BlockSpec((SEQ, UNPACK_P, 128), lambda i: (0, i, 0))],
        out_specs=pl.BlockSpec(
            (SEQ, EMBED_DIM, 2 * UNPACK_P), lambda i: (0, 0, i)
        ),
        out_shape=jax.ShapeDtypeStruct((SEQ, EMBED_DIM, BATCH), jnp.float32),
    )(packed)


def kernel(action_indices, table, W1, b1, W2, b2):
    idxt = action_indices.astype(jnp.int32).T
    fused = _fuse_table(table, W1, b1, W2, b2)
    return jnp.transpose(_unpack(_sc_gather(fused, idxt)), (2, 0, 1))


# speedup vs baseline: 1.0882x; 1.0370x over previous
import functools

import jax
import jax.numpy as jnp
from jax import lax
from jax.experimental import pallas as pl
from jax.experimental.pallas import tpu as pltpu
from jax.experimental.pallas import tpu_sc as plsc

NUM_ACTIONS = 100000
EMBED_DIM = 64
HIDDEN_DIM = 256
BATCH = 16384
SEQ = 50

ROW_BLOCK = 4000

NC = 2
NS = 16
NW = NC * NS
B_PER_W = BATCH // NW
CHUNK = 128
CHUNKS_PER_S = B_PER_W // CHUNK
N_CHUNKS = SEQ * CHUNKS_PER_S
PACK_B = BATCH // 2


def _mlp_block(table_ref, w1_ref, b1_ref, w2_ref, b2_ref, out_ref):
    t = table_ref[...].astype(jnp.bfloat16)
    h = (
        jnp.dot(t, w1_ref[...].astype(jnp.bfloat16), preferred_element_type=jnp.float32)
        + b1_ref[...]
    )
    h = h * jax.nn.sigmoid(h)
    out_ref[...] = (
        jnp.dot(
            h.astype(jnp.bfloat16),
            w2_ref[...].astype(jnp.bfloat16),
            preferred_element_type=jnp.float32,
        )
        + b2_ref[...]
    )


def _fuse_table(table, W1, b1, W2, b2):
    grid = (NUM_ACTIONS // ROW_BLOCK,)
    return pl.pallas_call(
        _mlp_block,
        grid=grid,
        in_specs=[
            pl.BlockSpec((ROW_BLOCK, HIDDEN_DIM), lambda i: (i, 0)),
            pl.BlockSpec((HIDDEN_DIM, HIDDEN_DIM), lambda i: (0, 0)),
            pl.BlockSpec((1, HIDDEN_DIM), lambda i: (0, 0)),
            pl.BlockSpec((HIDDEN_DIM, EMBED_DIM), lambda i: (0, 0)),
            pl.BlockSpec((1, EMBED_DIM), lambda i: (0, 0)),
        ],
        out_specs=pl.BlockSpec((ROW_BLOCK, EMBED_DIM), lambda i: (i, 0)),
        out_shape=jax.ShapeDtypeStruct((NUM_ACTIONS, EMBED_DIM), jnp.float32),
    )(table, W1, b1.reshape(1, HIDDEN_DIM), W2, b2.reshape(1, EMBED_DIM))


def _gather_body(
    fused_hbm, idxt_hbm, out_hbm, idx_v, rows_v, sem_a, sem_b, sem_c, sem_d
):
    wid = lax.axis_index("s") * NC + lax.axis_index("c")
    b0 = wid * B_PER_W
    pltpu.sync_copy(idxt_hbm.at[:, pl.ds(b0, B_PER_W)], idx_v)

    def fire(c, slot, sem):
        s = c // CHUNKS_PER_S
        j = lax.rem(c, CHUNKS_PER_S)
        pltpu.async_copy(
            fused_hbm.at[idx_v.at[s, pl.ds(j * CHUNK, CHUNK)]],
            rows_v.at[slot],
            sem,
        )

    def drain_and_write(c, slot, sem):
        pltpu.make_async_copy(
            fused_hbm.at[idx_v.at[0, pl.ds(0, CHUNK)]], rows_v.at[slot], sem
        ).wait()
        s = c // CHUNKS_PER_S
        j = lax.rem(c, CHUNKS_PER_S)
        p0 = wid * (B_PER_W // 2) + lax.rem(j, 2) * CHUNK
        col = (j // 2) * EMBED_DIM
        pltpu.sync_copy(
            rows_v.at[slot],
            out_hbm.at[s, pl.ds(p0, CHUNK), pl.ds(col, EMBED_DIM)],
        )

    sems = (sem_a, sem_b, sem_c, sem_d)
    fire(0, 0, sems[0])
    fire(1, 1, sems[1])

    def step(c, _):
        phase = lax.rem(c, 4)

        for k in range(4):

            @pl.when(jnp.logical_and(phase == k, c + 2 < N_CHUNKS))
            def _(k=k):
                fire(c + 2, (k + 2) % 4, sems[(k + 2) % 4])

        for k in range(4):

            @pl.when(phase == k)
            def _(k=k):
                drain_and_write(c, k, sems[k])

        return 0

    lax.fori_loop(0, N_CHUNKS, step, 0)


@jax.jit
def _sc_gather(fused, idxt):
    mesh = plsc.VectorSubcoreMesh(core_axis_name="c", subcore_axis_name="s")
    return pl.kernel(
        _gather_body,
        out_type=jax.ShapeDtypeStruct((SEQ, PACK_B, 2 * EMBED_DIM), jnp.float32),
        mesh=mesh,
        compiler_params=pltpu.CompilerParams(use_tc_tiling_on_sc=False),
        scratch_types=[
            pltpu.VMEM((SEQ, B_PER_W), jnp.int32),
            pltpu.VMEM((4, CHUNK, EMBED_DIM), jnp.float32),
            pltpu.SemaphoreType.DMA,
            pltpu.SemaphoreType.DMA,
            pltpu.SemaphoreType.DMA,
            pltpu.SemaphoreType.DMA,
        ],
    )(fused, idxt)


UNPACK_P = 256


def _unpack_block(packed_ref, out_ref):
    x = packed_ref[...]
    lo = x[:, :, :EMBED_DIM].transpose(0, 2, 1)
    hi = x[:, :, EMBED_DIM:].transpose(0, 2, 1)
    out_ref[...] = jnp.concatenate([lo, hi], axis=2)


def _unpack(packed):
    return pl.pallas_call(
        _unpack_block,
        grid=(PACK_B // UNPACK_P,),
        in_specs=[pl.BlockSpec((SEQ, UNPACK_P, 128), lambda i: (0, i, 0))],
        out_specs=pl.BlockSpec(
            (SEQ, EMBED_DIM, 2 * UNPACK_P), lambda i: (0, 0, i)
        ),
        out_shape=jax.ShapeDtypeStruct((SEQ, EMBED_DIM, BATCH), jnp.float32),
    )(packed)


def kernel(action_indices, table, W1, b1, W2, b2):
    idxt = action_indices.astype(jnp.int32).T
    fused = _fuse_table(table, W1, b1, W2, b2)
    return jnp.transpose(_unpack(_sc_gather(fused, idxt)), (2, 0, 1))
